# Initial kernel scaffold; baseline (speedup 1.0000x reference)
import jax
import jax.numpy as jnp
from jax.experimental import pallas as pl


def kernel(x, edge_index, W1, att_src1, att_dst1, b1, Ws1, bs1, W2, att_src2, att_dst2, b2, Ws2, bs2, W3, att_src3, att_dst3, b3, Ws3, bs3):
    raise NotImplementedError("write your pallas kernel here")


# speedup vs baseline: 6.8569x; 6.8569x over previous
import functools

import jax
import jax.numpy as jnp
from jax import lax
from jax.experimental import pallas as pl
from jax.experimental.pallas import tpu as pltpu
from jax.experimental.pallas import tpu_sc as plsc

N = 10000
NP = 10240
NBKT = 32
BKT = NP // NBKT
CAP = 5888
SL = 128
EB = 64
ECH = 1600


def _mesh():
    return plsc.VectorSubcoreMesh(core_axis_name="c", subcore_axis_name="s")


def _wid():
    return lax.axis_index("s") * 2 + lax.axis_index("c")


@functools.partial(
    pl.kernel,
    out_type=(
        jax.ShapeDtypeStruct((NBKT, CAP), jnp.int32),
        jax.ShapeDtypeStruct((NBKT, CAP), jnp.int32),
        jax.ShapeDtypeStruct((NBKT, 16), jnp.int32),
    ),
    mesh=_mesh(),
    compiler_params=pltpu.CompilerParams(needs_layout_passes=False),
    scratch_types=(
        pltpu.VMEM((ECH,), jnp.int32),
        pltpu.VMEM((ECH,), jnp.int32),
        pltpu.VMEM((CAP,), jnp.int32),
        pltpu.VMEM((CAP,), jnp.int32),
        pltpu.VMEM((16,), jnp.int32),
    ),
)
def _bucket_kernel(src_hbm, dst_hbm, osrc, odst, ocnt, sbuf, dbuf, slist,
                   dlist, cvec):
    wid = _wid()
    base = wid * BKT
    zero16 = jnp.zeros((16,), jnp.int32)

    def zbody(i, _):
        slist[pl.ds(i * 16, 16)] = zero16
        dlist[pl.ds(i * 16, 16)] = zero16
        return 0

    lax.fori_loop(0, CAP // 16, zbody, 0)

    nch = src_hbm.shape[0] // ECH

    def cbody(c, off):
        pltpu.sync_copy(src_hbm.at[pl.ds(c * ECH, ECH)], sbuf)
        pltpu.sync_copy(dst_hbm.at[pl.ds(c * ECH, ECH)], dbuf)

        def ibody(i, off):
            d16 = dbuf[pl.ds(i * 16, 16)]
            s16 = sbuf[pl.ds(i * 16, 16)]
            m = (d16 >= base) & (d16 < base + BKT)
            mi = m.astype(jnp.int32)
            cs = plsc.cumsum(mi)
            pos = off + cs - mi
            plsc.store_scatter(dlist, [pos], d16 - base, mask=m)
            plsc.store_scatter(slist, [pos], s16, mask=m)
            return off + cs[15]

        return lax.fori_loop(0, ECH // 16, ibody, off)

    off = lax.fori_loop(0, nch, cbody, jnp.int32(0))
    cvec[...] = jnp.full((16,), off, jnp.int32)
    pltpu.sync_copy(slist, osrc.at[wid])
    pltpu.sync_copy(dlist, odst.at[wid])
    pltpu.sync_copy(cvec, ocnt.at[wid])


def _make_alpha_kernel(K):
    @functools.partial(
        pl.kernel,
        out_type=(
            jax.ShapeDtypeStruct((NBKT, CAP * K), jnp.float32),
            jax.ShapeDtypeStruct((NBKT, BKT * K), jnp.float32),
        ),
        mesh=_mesh(),
        compiler_params=pltpu.CompilerParams(needs_layout_passes=False),
        scratch_types=(
            pltpu.VMEM((K * NP,), jnp.float32),
            pltpu.VMEM((BKT * 2 * K,), jnp.float32),
            pltpu.VMEM((CAP,), jnp.int32),
            pltpu.VMEM((CAP,), jnp.int32),
            pltpu.VMEM((CAP * K,), jnp.float32),
            pltpu.VMEM((BKT * K,), jnp.float32),
            pltpu.VMEM((16,), jnp.int32),
        ),
    )
    def k_alpha(asrc_hbm, cdst_hbm, slists, dlists, cnts, w_out, den_out,
                asrc_v, cdst_v, sl, dl, wv, den, cv):
        wid = _wid()
        base = wid * BKT
        pltpu.sync_copy(asrc_hbm, asrc_v)
        pltpu.sync_copy(cdst_hbm.at[pl.ds(base * 2 * K, BKT * 2 * K)], cdst_v)
        pltpu.sync_copy(slists.at[wid], sl)
        pltpu.sync_copy(dlists.at[wid], dl)
        pltpu.sync_copy(cnts.at[wid], cv)
        cnt = cv[...][0]
        iota = lax.iota(jnp.int32, 16)
        iK = iota * K
        ones16 = jnp.full((16,), 1.0, jnp.float32)

        def dinit(i, _):
            den[pl.ds(i * 16, 16)] = ones16
            return 0

        lax.fori_loop(0, (BKT * K) // 16, dinit, 0)

        def body(i, _):
            off = i * 16
            s16 = sl[pl.ds(off, 16)]
            d16 = dl[pl.ds(off, 16)]
            m = (off + iota) < cnt
            for k in range(K):
                a_s = plsc.load_gather(asrc_v, [s16 * K + k])
                a_d = plsc.load_gather(cdst_v, [d16 * (2 * K) + k])
                e_s = plsc.load_gather(cdst_v, [d16 * (2 * K) + (K + k)])
                t = a_s + a_d
                al = jnp.where(t > 0, t, 0.2 * t)
                w = jnp.exp(jnp.minimum(al - e_s, 80.0))
                plsc.store_scatter(wv, [iK + (off * K + k)], w)
                plsc.addupdate_scatter(den, [d16 * K + k], w, mask=m)
            return 0

        lax.fori_loop(0, (cnt + 15) // 16, body, 0)
        pltpu.sync_copy(wv, w_out.at[wid])
        pltpu.sync_copy(den, den_out.at[wid])

    return k_alpha


def _make_agg_kernel(K):
    NSL = 2 * K

    @functools.partial(
        pl.kernel,
        out_type=jax.ShapeDtypeStruct((NSL * NP, SL), jnp.float32),
        mesh=_mesh(),
        compiler_params=pltpu.CompilerParams(needs_layout_passes=False),
        scratch_types=(
            pltpu.VMEM((CAP,), jnp.int32),
            pltpu.VMEM((CAP,), jnp.int32),
            pltpu.VMEM((CAP * K,), jnp.float32),
            pltpu.VMEM((16,), jnp.int32),
            pltpu.VMEM((BKT, SL), jnp.float32),
            pltpu.VMEM((EB, SL), jnp.float32),
            pltpu.VMEM((EB,), jnp.int32),
            pltpu.VMEM((BKT * K,), jnp.float32),
            pltpu.SemaphoreType.DMA,
        ),
    )
    def k_agg(hview, slists, dlists, cnts, wlists, denoms, out, sl, dl,
              wv, cv, acc, stage, idxv, den, sem):
        wid = _wid()
        base = wid * BKT
        pltpu.sync_copy(slists.at[wid], sl)
        pltpu.sync_copy(dlists.at[wid], dl)
        pltpu.sync_copy(wlists.at[wid], wv)
        pltpu.sync_copy(denoms.at[wid], den)
        pltpu.sync_copy(cnts.at[wid], cv)
        cnt = cv[...][0]
        iota = lax.iota(jnp.int32, 16)
        nb = (cnt + EB - 1) // EB

        def pass_body(p, _):
            k = p // 2
            pltpu.sync_copy(hview.at[pl.ds(p * NP + base, BKT)], acc)
            pNp = p * NP

            def bat_body(b, _):
                e0 = b * EB
                for c4 in range(EB // 16):
                    s16 = sl[pl.ds(e0 + c4 * 16, 16)]
                    idxv[pl.ds(c4 * 16, 16)] = s16 + pNp
                pltpu.async_copy(hview.at[idxv], stage, sem).wait()
                for c4 in range(EB // 16):
                    ebase = e0 + c4 * 16
                    d16 = dl[pl.ds(ebase, 16)]
                    w16 = plsc.load_gather(wv, [(ebase + iota) * K + k])
                    w16 = jnp.where((ebase + iota) < cnt, w16, 0.0)
                    for e in range(16):
                        d = d16[e]
                        wvv = w16[e]
                        row = c4 * 16 + e
                        for c in range(SL // 16):
                            v = stage[row, pl.ds(c * 16, 16)]
                            plsc.addupdate(acc.at[d, pl.ds(c * 16, 16)],
                                           wvv * v)
                return 0

            lax.fori_loop(0, nb, bat_body, 0)

            def rbody(i, _):
                r0 = i * 16
                den16 = plsc.load_gather(den, [(r0 + iota) * K + k])
                inv16 = 1.0 / den16
                for e in range(16):
                    iv = inv16[e]
                    r = r0 + e
                    for c in range(SL // 16):
                        sli = pl.ds(c * 16, 16)
                        acc[r, sli] = acc[r, sli] * iv
                return 0

            lax.fori_loop(0, BKT // 16, rbody, 0)
            pltpu.sync_copy(acc, out.at[pl.ds(p * NP + base, BKT)])
            return 0

        lax.fori_loop(0, NSL, pass_body, 0)

    return k_agg


def _prologue(x, W, att_s, att_d, K, Din):
    NSL = 2 * K
    CH = K * 256

    def body(x_ref, w_ref, as_ref, ad_ref, ht_ref, asr_ref, adr_ref, esr_ref):
        h = jnp.dot(x_ref[...], w_ref[...],
                    preferred_element_type=jnp.float32)
        h3 = h.reshape(256, K, 256)
        asv = jnp.sum(h3 * as_ref[...][None, :, :], axis=2)
        adv = jnp.sum(h3 * ad_ref[...][None, :, :], axis=2)
        t = asv + adv
        es = jnp.where(t > 0, t, 0.2 * t)
        ht_ref[...] = h.reshape(256, NSL, 128).transpose(1, 0, 2)
        asr_ref[...] = asv
        adr_ref[...] = adv
        esr_ref[...] = es

    return pl.pallas_call(
        body,
        grid=(NP // 256,),
        in_specs=[
            pl.BlockSpec((256, Din), lambda i: (i, 0)),
            pl.BlockSpec((Din, CH), lambda i: (0, 0)),
            pl.BlockSpec((K, 256), lambda i: (0, 0)),
            pl.BlockSpec((K, 256), lambda i: (0, 0)),
        ],
        out_specs=[
            pl.BlockSpec((NSL, 256, 128), lambda i: (0, i, 0)),
            pl.BlockSpec((256, K), lambda i: (i, 0)),
            pl.BlockSpec((256, K), lambda i: (i, 0)),
            pl.BlockSpec((256, K), lambda i: (i, 0)),
        ],
        out_shape=(
            jax.ShapeDtypeStruct((NSL, NP, 128), jnp.float32),
            jax.ShapeDtypeStruct((NP, K), jnp.float32),
            jax.ShapeDtypeStruct((NP, K), jnp.float32),
            jax.ShapeDtypeStruct((NP, K), jnp.float32),
        ),
    )(x, W, att_s, att_d)


def _matmul(x, W):
    Din, Cout = W.shape

    def body(x_ref, w_ref, o_ref):
        o_ref[...] = jnp.dot(x_ref[...], w_ref[...],
                             preferred_element_type=jnp.float32)

    return pl.pallas_call(
        body,
        grid=(NP // 256,),
        in_specs=[
            pl.BlockSpec((256, Din), lambda i: (i, 0)),
            pl.BlockSpec((Din, Cout), lambda i: (0, 0)),
        ],
        out_specs=pl.BlockSpec((256, Cout), lambda i: (i, 0)),
        out_shape=jax.ShapeDtypeStruct((NP, Cout), jnp.float32),
    )(x, W)


def _combine12(aggt, skip, bsum, K):
    NSL = 2 * K
    CH = K * 256

    def body(a_ref, s_ref, b_ref, o_ref):
        g = a_ref[...].transpose(1, 0, 2).reshape(256, CH)
        y = g + s_ref[...] + b_ref[...]
        o_ref[...] = jnp.where(y > 0, y, jnp.exp(jnp.minimum(y, 0.0)) - 1.0)

    return pl.pallas_call(
        body,
        grid=(NP // 256,),
        in_specs=[
            pl.BlockSpec((NSL, 256, 128), lambda i: (0, i, 0)),
            pl.BlockSpec((256, CH), lambda i: (i, 0)),
            pl.BlockSpec((1, CH), lambda i: (0, 0)),
        ],
        out_specs=pl.BlockSpec((256, CH), lambda i: (i, 0)),
        out_shape=jax.ShapeDtypeStruct((NP, CH), jnp.float32),
    )(aggt, skip, bsum)


def _combine3(aggt, skip, bsum):
    def body(a_ref, s_ref, b_ref, o_ref):
        g = a_ref[...].transpose(1, 0, 2).reshape(256, 6 * 256)
        acc = g[:, 0:256]
        for h in range(1, 6):
            acc = acc + g[:, h * 256:(h + 1) * 256]
        o_ref[...] = acc * (1.0 / 6.0) + s_ref[...] + b_ref[...]

    return pl.pallas_call(
        body,
        grid=(NP // 256,),
        in_specs=[
            pl.BlockSpec((12, 256, 128), lambda i: (0, i, 0)),
            pl.BlockSpec((256, 256), lambda i: (i, 0)),
            pl.BlockSpec((1, 256), lambda i: (0, 0)),
        ],
        out_specs=pl.BlockSpec((256, 256), lambda i: (i, 0)),
        out_shape=jax.ShapeDtypeStruct((NP, 256), jnp.float32),
    )(aggt, skip, bsum)


def _gat_layer(h, W, att_s, att_d, b, Ws, bs, K, lists):
    slists, dlists, cnts = lists
    NSL = 2 * K
    Din = h.shape[1]
    ht, asr, adr, esr = _prologue(h, W, att_s.reshape(K, 256),
                                  att_d.reshape(K, 256), K, Din)
    skip = _matmul(h, Ws)
    cdst = jnp.concatenate([adr, esr], axis=1).reshape(-1)
    w, den = _make_alpha_kernel(K)(asr.reshape(-1), cdst, slists, dlists,
                                   cnts)
    aggt = _make_agg_kernel(K)(ht.reshape(NSL * NP, SL),
                               slists, dlists, cnts, w, den)
    aggt = aggt.reshape(NSL, NP, SL)
    if K == 4:
        return _combine12(aggt, skip, (b + bs).reshape(1, K * 256), K)
    return _combine3(aggt, skip, (b + bs).reshape(1, 256))


def kernel(x, edge_index, W1, att_src1, att_dst1, b1, Ws1, bs1,
           W2, att_src2, att_dst2, b2, Ws2, bs2,
           W3, att_src3, att_dst3, b3, Ws3, bs3):
    xp = jnp.pad(x, ((0, NP - N), (0, 0)))
    lists = _bucket_kernel(edge_index[0], edge_index[1])
    h = _gat_layer(xp, W1, att_src1, att_dst1, b1, Ws1, bs1, 4, lists)
    h = _gat_layer(h, W2, att_src2, att_dst2, b2, Ws2, bs2, 4, lists)
    out = _gat_layer(h, W3, att_src3, att_dst3, b3, Ws3, bs3, 6, lists)
    return out[:N]
